# baseline (speedup 1.0000x reference)
import functools

import jax
import jax.numpy as jnp
import numpy as np
from jax.experimental import pallas as pl
from jax.experimental.pallas import tpu as pltpu
from jax.sharding import Mesh, PartitionSpec as P

BD = 1024
I_ = 2048
NS = 64
NP = 128
H_ = 16
P_ = 128
CONV = 2176
CHUNK = 256
F32 = jnp.float32
BF16 = jnp.bfloat16


def _matmul_bias_kernel(x_ref, w_ref, b_ref, o_ref):
    o_ref[...] = jnp.dot(x_ref[...].astype(BF16), w_ref[...],
                         preferred_element_type=F32) + b_ref[...]


def _matmul_bias(x, w, b, block_m, name):
    m, k = x.shape
    n = w.shape[1]
    return pl.pallas_call(
        _matmul_bias_kernel,
        out_shape=jax.ShapeDtypeStruct((m, n), F32),
        grid=(m // block_m,),
        in_specs=[
            pl.BlockSpec((block_m, k), lambda i: (i, 0)),
            pl.BlockSpec((k, n), lambda i: (0, 0)),
            pl.BlockSpec((1, n), lambda i: (0, 0)),
        ],
        out_specs=pl.BlockSpec((block_m, n), lambda i: (i, 0)),
        compiler_params=pltpu.CompilerParams(
            dimension_semantics=("parallel",),
            vmem_limit_bytes=50 * 1024 * 1024,
        ),
        name=name,
    )(x, w, b)


def _layer_kernel(h_ref, wall_ref, cw_ref, cb_ref,
                  dtb_ref, alog_ref, dv_ref, gnw_ref, nw_ref, outw_ref,
                  ones_ref, ho_ref, state_ref, halo_ref, yscr_ref):
    c = pl.program_id(1)
    C = CHUNK

    @pl.when(c == 0)
    def _():
        state_ref[...] = jnp.zeros_like(state_ref)
        halo_ref[...] = jnp.zeros_like(halo_ref)

    h = h_ref[0]
    sq = (h * h).astype(BF16)
    v = jnp.dot(sq, ones_ref[:BD, :], preferred_element_type=F32)
    rs = jax.lax.rsqrt(v * (1.0 / BD) + 1e-6)
    hn = (h * pltpu.repeat(rs, BD // 128, axis=1) * nw_ref[...]).astype(BF16)

    proj = jnp.dot(hn, wall_ref[...], preferred_element_type=F32)
    gate = proj[:, :I_]
    xbc = proj[:, I_:I_ + CONV]
    dtr = proj[:, I_ + CONV:]

    prev = halo_ref[0:2, :]
    x2 = jnp.concatenate([prev, xbc[:C - 2]], axis=0)
    x1 = jnp.concatenate([prev[1:2], xbc[:C - 1]], axis=0)
    conv = x2 * cw_ref[0:1] + x1 * cw_ref[1:2] + xbc * cw_ref[2:3] + cb_ref[...]
    halo_ref[0:2, :] = xbc[C - 2:C]
    conv = conv * jax.nn.sigmoid(conv)

    xs = conv[:, :I_]
    BCt = conv[:, I_:]
    lane = jax.lax.broadcasted_iota(jnp.int32, (C, NP), 1)
    Bp = jnp.where(lane < NS, BCt, 0.0)
    Crot = jnp.concatenate([BCt[:, NS:], BCt[:, :NS]], axis=1)
    Cp = jnp.where(lane < NS, Crot, 0.0)
    Bpb = Bp.astype(BF16)
    Cpb = Cp.astype(BF16)

    G = jax.lax.dot_general(Cpb, Bpb, (((1,), (1,)), ((), ())),
                            preferred_element_type=F32)

    dt = jax.nn.softplus(dtr + dtb_ref[...])
    a = -jnp.exp(alog_ref[...])
    al = dt * a
    s = al
    k = 1
    while k < C:
        s = s + jnp.concatenate([jnp.zeros((k, H_), F32), s[:C - k]], axis=0)
        k *= 2
    ES = jnp.exp(s)
    EMS = jnp.exp(-s)
    EMT = EMS.T
    mask = (jax.lax.broadcasted_iota(jnp.int32, (C, C), 0)
            >= jax.lax.broadcasted_iota(jnp.int32, (C, C), 1))

    for hh in range(H_):
        Xh = xs[:, hh * P_:(hh + 1) * P_]
        dth = dt[:, hh:hh + 1]
        esi = ES[:, hh:hh + 1]
        elast = ES[C - 1:C, hh:hh + 1]
        SG = jnp.where(mask, G * EMT[hh:hh + 1, :], 0.0).astype(BF16)
        Xdt = (Xh * dth).astype(BF16)
        ST = state_ref[hh]
        yin = jnp.dot(SG, Xdt, preferred_element_type=F32)
        yin = yin + jnp.dot(Cpb, ST.astype(BF16), preferred_element_type=F32)
        y = esi * yin + dv_ref[0:1, hh:hh + 1] * Xh
        scl = EMS[:, hh:hh + 1] * elast * dth
        Xs = (Xh * scl).astype(BF16)
        state_ref[hh] = ST * elast + jax.lax.dot_general(
            Bpb, Xs, (((0,), (0,)), ((), ())), preferred_element_type=F32)
        yscr_ref[:, hh * P_:(hh + 1) * P_] = y
    yf = yscr_ref[...]

    yg = yf * (gate * jax.nn.sigmoid(gate))
    sq2 = (yg * yg).astype(BF16)
    vv = jnp.dot(sq2, ones_ref[...], preferred_element_type=F32)
    rs2 = jax.lax.rsqrt(vv * (1.0 / I_) + 1e-6)
    yn = (yg * pltpu.repeat(rs2, I_ // 128, axis=1) * gnw_ref[...]).astype(BF16)
    out = jnp.dot(yn, outw_ref[...], preferred_element_type=F32)
    ho_ref[0] = h + out


def _layer(h, wall, cw, cb, dtb, alog, dv, gnw, nw, outw, ones, name):
    Bb, T, _ = h.shape
    nc = T // CHUNK
    full = lambda arr: pl.BlockSpec(arr.shape, lambda b, c: (0,) * arr.ndim)
    return pl.pallas_call(
        _layer_kernel,
        out_shape=jax.ShapeDtypeStruct((Bb, T, BD), F32),
        grid=(Bb, nc),
        in_specs=[
            pl.BlockSpec((1, CHUNK, BD), lambda b, c: (b, c, 0)),
            full(wall), full(cw), full(cb),
            full(dtb), full(alog), full(dv), full(gnw), full(nw), full(outw),
            full(ones),
        ],
        out_specs=pl.BlockSpec((1, CHUNK, BD), lambda b, c: (b, c, 0)),
        scratch_shapes=[
            pltpu.VMEM((H_, NP, P_), F32),
            pltpu.VMEM((8, CONV), F32),
            pltpu.VMEM((CHUNK, I_), F32),
        ],
        compiler_params=pltpu.CompilerParams(
            dimension_semantics=("parallel", "arbitrary"),
            vmem_limit_bytes=50 * 1024 * 1024,
        ),
        name=name,
    )(h, wall, cw, cb, dtb, alog, dv, gnw, nw, outw, ones)


def _forward(x, in_w, in_b, norm_w, mix_in_w, conv_w, conv_b, dt_bias,
             A_log, D, gnorm_w, mix_out_w, out_w, out_b, code_w, code_b):
    Bb, T, IN = x.shape
    L = mix_in_w.shape[0]

    h = _matmul_bias(x.reshape(Bb * T, IN), in_w.astype(BF16),
                     in_b.reshape(1, BD), 1024, "in_proj")
    h = h.reshape(Bb, T, BD)

    ones = jnp.ones((I_, 128), BF16)
    wmix = mix_in_w.astype(BF16)
    wout = mix_out_w.astype(BF16)
    for l in range(L):
        h = _layer(
            h,
            wmix[l],
            conv_w[l][:, 0, :],
            conv_b[l].reshape(1, CONV),
            dt_bias[l].reshape(1, H_),
            A_log[l].reshape(1, H_),
            D[l].reshape(1, H_),
            gnorm_w[l].reshape(1, I_),
            norm_w[l].reshape(1, BD),
            wout[l],
            ones,
            f"mamba_layer_{l}",
        )

    wcat = jnp.concatenate([out_w, code_w], axis=1).astype(BF16)
    bcat = jnp.concatenate([out_b, code_b]).reshape(1, -1)
    o = _matmul_bias(h.reshape(Bb * T, BD), wcat, bcat, 1024, "out_heads")
    no = out_w.shape[1]
    return (o[:, :no].reshape(Bb, T, no),
            o[:, no:].reshape(Bb, T, code_w.shape[1]))


def kernel(x, in_w, in_b, norm_w, mix_in_w, conv_w, conv_b, dt_bias,
           A_log, D, gnorm_w, mix_out_w, out_w, out_b, code_w, code_b):
    args = (x, in_w, in_b, norm_w, mix_in_w, conv_w, conv_b, dt_bias,
            A_log, D, gnorm_w, mix_out_w, out_w, out_b, code_w, code_b)
    devs = jax.devices()
    if len(devs) < 2 or x.shape[0] % 2 != 0:
        return _forward(*args)
    mesh = Mesh(np.array(devs[:2]), ("d",))
    in_specs = (P("d"),) + (P(),) * 15
    out_specs = (P("d"), P("d"))
    f = jax.shard_map(_forward, mesh=mesh, in_specs=in_specs,
                      out_specs=out_specs, check_vma=False)
    return f(*args)


# speedup vs baseline: 1.0217x; 1.0217x over previous
import jax
import jax.numpy as jnp
from jax.experimental import pallas as pl
from jax.experimental.pallas import tpu as pltpu

BD = 1024
I_ = 2048
NS = 64
NP = 128
H_ = 16
P_ = 128
CONV = 2176
CHUNK = 256
PB = 2
F32 = jnp.float32
BF16 = jnp.bfloat16


def _matmul_bias_kernel(x_ref, w_ref, b_ref, o_ref):
    o_ref[...] = jnp.dot(x_ref[...].astype(BF16), w_ref[...],
                         preferred_element_type=F32) + b_ref[...]


def _matmul_bias(x, w, b, block_m, name):
    m, k = x.shape
    n = w.shape[1]
    return pl.pallas_call(
        _matmul_bias_kernel,
        out_shape=jax.ShapeDtypeStruct((m, n), F32),
        grid=(m // block_m,),
        in_specs=[
            pl.BlockSpec((block_m, k), lambda i: (i, 0)),
            pl.BlockSpec((k, n), lambda i: (0, 0)),
            pl.BlockSpec((1, n), lambda i: (0, 0)),
        ],
        out_specs=pl.BlockSpec((block_m, n), lambda i: (i, 0)),
        compiler_params=pltpu.CompilerParams(
            dimension_semantics=("parallel",),
            vmem_limit_bytes=50 * 1024 * 1024,
        ),
        name=name,
    )(x, w, b)


def _layer_kernel(h_ref, wall_ref, cw_ref, cb_ref,
                  dtb_ref, alog_ref, dv_ref, gnw_ref, nw_ref, outw_ref,
                  ones_ref, ho_ref, state_ref, halo_ref, yscr_ref):
    c = pl.program_id(1)
    C = CHUNK

    @pl.when(c == 0)
    def _():
        state_ref[...] = jnp.zeros_like(state_ref)
        halo_ref[...] = jnp.zeros_like(halo_ref)

    mask = (jax.lax.broadcasted_iota(jnp.int32, (C, C), 0)
            >= jax.lax.broadcasted_iota(jnp.int32, (C, C), 1))
    lane = jax.lax.broadcasted_iota(jnp.int32, (C, NP), 1)

    for bb in range(PB):
        h = h_ref[bb]
        sq = (h * h).astype(BF16)
        v = jnp.dot(sq, ones_ref[:BD, :], preferred_element_type=F32)
        rs = jax.lax.rsqrt(v * (1.0 / BD) + 1e-6)
        hn = (h * pltpu.repeat(rs, BD // 128, axis=1) * nw_ref[...]).astype(BF16)

        proj = jnp.dot(hn, wall_ref[...], preferred_element_type=F32)
        gate = proj[:, :I_]
        xbc = proj[:, I_:I_ + CONV]
        dtr = proj[:, I_ + CONV:]

        prev = halo_ref[bb, 0:2, :]
        x2 = jnp.concatenate([prev, xbc[:C - 2]], axis=0)
        x1 = jnp.concatenate([prev[1:2], xbc[:C - 1]], axis=0)
        conv = (x2 * cw_ref[0:1] + x1 * cw_ref[1:2] + xbc * cw_ref[2:3]
                + cb_ref[...])
        halo_ref[bb, 0:2, :] = xbc[C - 2:C]
        conv = conv * jax.nn.sigmoid(conv)

        xs = conv[:, :I_]
        BCt = conv[:, I_:]
        Bp = jnp.where(lane < NS, BCt, 0.0)
        Crot = jnp.concatenate([BCt[:, NS:], BCt[:, :NS]], axis=1)
        Cp = jnp.where(lane < NS, Crot, 0.0)
        Bpb = Bp.astype(BF16)
        Cpb = Cp.astype(BF16)

        G = jax.lax.dot_general(Cpb, Bpb, (((1,), (1,)), ((), ())),
                                preferred_element_type=F32)

        dt = jax.nn.softplus(dtr + dtb_ref[...])
        a = -jnp.exp(alog_ref[...])
        al = dt * a
        s = al
        k = 1
        while k < C:
            s = s + jnp.concatenate([jnp.zeros((k, H_), F32), s[:C - k]],
                                    axis=0)
            k *= 2
        ES = jnp.exp(s)
        EMS = jnp.exp(-s)
        EMT = EMS.T

        for hh in range(H_):
            Xh = xs[:, hh * P_:(hh + 1) * P_]
            dth = dt[:, hh:hh + 1]
            esi = ES[:, hh:hh + 1]
            elast = ES[C - 1:C, hh:hh + 1]
            SG = jnp.where(mask, G * EMT[hh:hh + 1, :], 0.0).astype(BF16)
            Xdt = (Xh * dth).astype(BF16)
            ST = state_ref[bb, hh]
            yin = jnp.dot(SG, Xdt, preferred_element_type=F32)
            yin = yin + jnp.dot(Cpb, ST.astype(BF16),
                                preferred_element_type=F32)
            y = esi * yin + dv_ref[0:1, hh:hh + 1] * Xh
            scl = EMS[:, hh:hh + 1] * elast * dth
            Xs = (Xh * scl).astype(BF16)
            state_ref[bb, hh] = ST * elast + jax.lax.dot_general(
                Bpb, Xs, (((0,), (0,)), ((), ())), preferred_element_type=F32)
            yscr_ref[bb, :, hh * P_:(hh + 1) * P_] = y
        yf = yscr_ref[bb]

        yg = yf * (gate * jax.nn.sigmoid(gate))
        sq2 = (yg * yg).astype(BF16)
        vv = jnp.dot(sq2, ones_ref[...], preferred_element_type=F32)
        rs2 = jax.lax.rsqrt(vv * (1.0 / I_) + 1e-6)
        yn = (yg * pltpu.repeat(rs2, I_ // 128, axis=1)
              * gnw_ref[...]).astype(BF16)
        out = jnp.dot(yn, outw_ref[...], preferred_element_type=F32)
        ho_ref[bb] = h + out


def _layer(h, wall, cw, cb, dtb, alog, dv, gnw, nw, outw, ones, name):
    Bb, T, _ = h.shape
    nc = T // CHUNK
    full = lambda arr: pl.BlockSpec(arr.shape, lambda b, c: (0,) * arr.ndim)
    return pl.pallas_call(
        _layer_kernel,
        out_shape=jax.ShapeDtypeStruct((Bb, T, BD), F32),
        grid=(Bb // PB, nc),
        in_specs=[
            pl.BlockSpec((PB, CHUNK, BD), lambda b, c: (b, c, 0)),
            full(wall), full(cw), full(cb),
            full(dtb), full(alog), full(dv), full(gnw), full(nw), full(outw),
            full(ones),
        ],
        out_specs=pl.BlockSpec((PB, CHUNK, BD), lambda b, c: (b, c, 0)),
        scratch_shapes=[
            pltpu.VMEM((PB, H_, NP, P_), F32),
            pltpu.VMEM((PB, 8, CONV), F32),
            pltpu.VMEM((PB, CHUNK, I_), F32),
        ],
        compiler_params=pltpu.CompilerParams(
            dimension_semantics=("parallel", "arbitrary"),
            vmem_limit_bytes=55 * 1024 * 1024,
        ),
        name=name,
    )(h, wall, cw, cb, dtb, alog, dv, gnw, nw, outw, ones)


def kernel(x, in_w, in_b, norm_w, mix_in_w, conv_w, conv_b, dt_bias,
           A_log, D, gnorm_w, mix_out_w, out_w, out_b, code_w, code_b):
    Bb, T, IN = x.shape
    L = mix_in_w.shape[0]

    h = _matmul_bias(x.reshape(Bb * T, IN), in_w.astype(BF16),
                     in_b.reshape(1, BD), 1024, "in_proj")
    h = h.reshape(Bb, T, BD)

    ones = jnp.ones((I_, 128), BF16)
    wmix = mix_in_w.astype(BF16)
    wout = mix_out_w.astype(BF16)
    for l in range(L):
        h = _layer(
            h,
            wmix[l],
            conv_w[l][:, 0, :],
            conv_b[l].reshape(1, CONV),
            dt_bias[l].reshape(1, H_),
            A_log[l].reshape(1, H_),
            D[l].reshape(1, H_),
            gnorm_w[l].reshape(1, I_),
            norm_w[l].reshape(1, BD),
            wout[l],
            ones,
            f"mamba_layer_{l}",
        )

    wcat = jnp.concatenate([out_w, code_w], axis=1).astype(BF16)
    bcat = jnp.concatenate([out_b, code_b]).reshape(1, -1)
    o = _matmul_bias(h.reshape(Bb * T, BD), wcat, bcat, 1024, "out_heads")
    no = out_w.shape[1]
    return (o[:, :no].reshape(Bb, T, no),
            o[:, no:].reshape(Bb, T, code_w.shape[1]))


# speedup vs baseline: 1.0547x; 1.0323x over previous
import functools

import jax
import jax.numpy as jnp
from jax.experimental import pallas as pl
from jax.experimental.pallas import tpu as pltpu

BD = 1024
I_ = 2048
NS = 64
NP = 128
H_ = 16
P_ = 128
CONV = 2176
CHUNK = 256
F32 = jnp.float32
BF16 = jnp.bfloat16


def _matmul_bias_kernel(x_ref, w_ref, b_ref, o_ref):
    o_ref[...] = jnp.dot(x_ref[...].astype(BF16), w_ref[...],
                         preferred_element_type=F32) + b_ref[...]


def _matmul_bias(x, w, b, block_m, name):
    m, k = x.shape
    n = w.shape[1]
    return pl.pallas_call(
        _matmul_bias_kernel,
        out_shape=jax.ShapeDtypeStruct((m, n), F32),
        grid=(m // block_m,),
        in_specs=[
            pl.BlockSpec((block_m, k), lambda i: (i, 0)),
            pl.BlockSpec((k, n), lambda i: (0, 0)),
            pl.BlockSpec((1, n), lambda i: (0, 0)),
        ],
        out_specs=pl.BlockSpec((block_m, n), lambda i: (i, 0)),
        compiler_params=pltpu.CompilerParams(
            dimension_semantics=("parallel",),
            vmem_limit_bytes=50 * 1024 * 1024,
        ),
        name=name,
    )(x, w, b)


def _layer_kernel(fuse_in, fuse_out, h_ref, inw_ref, inb_ref,
                  wall_ref, cw_ref, cb_ref,
                  dtb_ref, alog_ref, dv_ref, gnw_ref, nw_ref, outw_ref,
                  wcat_ref, bcat_ref,
                  ones_ref, ho_ref, state_ref, halo_ref, yscr_ref):
    c = pl.program_id(1)
    C = CHUNK

    @pl.when(c == 0)
    def _():
        state_ref[...] = jnp.zeros_like(state_ref)
        halo_ref[...] = jnp.zeros_like(halo_ref)

    if fuse_in:
        h = (jnp.dot(h_ref[0].astype(BF16), inw_ref[...],
                     preferred_element_type=F32) + inb_ref[...])
    else:
        h = h_ref[0]
    sq = (h * h).astype(BF16)
    v = jnp.dot(sq, ones_ref[:BD, :], preferred_element_type=F32)
    rs = jax.lax.rsqrt(v * (1.0 / BD) + 1e-6)
    hn = (h * pltpu.repeat(rs, BD // 128, axis=1) * nw_ref[...]).astype(BF16)

    proj = jnp.dot(hn, wall_ref[...], preferred_element_type=F32)
    gate = proj[:, :I_]
    xbc = proj[:, I_:I_ + CONV]
    dtr = proj[:, I_ + CONV:]

    prev = halo_ref[0:2, :]
    x2 = jnp.concatenate([prev, xbc[:C - 2]], axis=0)
    x1 = jnp.concatenate([prev[1:2], xbc[:C - 1]], axis=0)
    conv = x2 * cw_ref[0:1] + x1 * cw_ref[1:2] + xbc * cw_ref[2:3] + cb_ref[...]
    halo_ref[0:2, :] = xbc[C - 2:C]
    conv = conv * jax.nn.sigmoid(conv)

    xs = conv[:, :I_]
    BCt = conv[:, I_:]
    lane = jax.lax.broadcasted_iota(jnp.int32, (C, NP), 1)
    Bp = jnp.where(lane < NS, BCt, 0.0)
    Crot = jnp.concatenate([BCt[:, NS:], BCt[:, :NS]], axis=1)
    Cp = jnp.where(lane < NS, Crot, 0.0)
    Bpb = Bp.astype(BF16)
    Cpb = Cp.astype(BF16)

    G = jax.lax.dot_general(Cpb, Bpb, (((1,), (1,)), ((), ())),
                            preferred_element_type=F32)

    dt = jax.nn.softplus(dtr + dtb_ref[...])
    a = -jnp.exp(alog_ref[...])
    al = dt * a
    s = al
    k = 1
    while k < C:
        s = s + jnp.concatenate([jnp.zeros((k, H_), F32), s[:C - k]], axis=0)
        k *= 2
    ES = jnp.exp(s)
    EMS = jnp.exp(-s)
    EMT = EMS.T
    mask = (jax.lax.broadcasted_iota(jnp.int32, (C, C), 0)
            >= jax.lax.broadcasted_iota(jnp.int32, (C, C), 1))

    for hh in range(H_):
        Xh = xs[:, hh * P_:(hh + 1) * P_]
        dth = dt[:, hh:hh + 1]
        esi = ES[:, hh:hh + 1]
        elast = ES[C - 1:C, hh:hh + 1]
        SG = jnp.where(mask, G * EMT[hh:hh + 1, :], 0.0).astype(BF16)
        Xdt = (Xh * dth).astype(BF16)
        ST = state_ref[hh]
        yin = jnp.dot(SG, Xdt, preferred_element_type=F32)
        yin = yin + jnp.dot(Cpb, ST.astype(BF16), preferred_element_type=F32)
        y = esi * yin + dv_ref[0:1, hh:hh + 1] * Xh
        scl = EMS[:, hh:hh + 1] * elast * dth
        Xs = (Xh * scl).astype(BF16)
        state_ref[hh] = ST * elast + jax.lax.dot_general(
            Bpb, Xs, (((0,), (0,)), ((), ())), preferred_element_type=F32)
        yscr_ref[:, hh * P_:(hh + 1) * P_] = y
    yf = yscr_ref[...]

    yg = yf * (gate * jax.nn.sigmoid(gate))
    sq2 = (yg * yg).astype(BF16)
    vv = jnp.dot(sq2, ones_ref[...], preferred_element_type=F32)
    rs2 = jax.lax.rsqrt(vv * (1.0 / I_) + 1e-6)
    yn = (yg * pltpu.repeat(rs2, I_ // 128, axis=1) * gnw_ref[...]).astype(BF16)
    out = jnp.dot(yn, outw_ref[...], preferred_element_type=F32)
    ho = h + out
    if fuse_out:
        ho_ref[0] = (jnp.dot(ho.astype(BF16), wcat_ref[...],
                             preferred_element_type=F32) + bcat_ref[...])
    else:
        ho_ref[0] = ho


def _layer(h, inw, inb, wall, cw, cb, dtb, alog, dv, gnw, nw, outw,
           wcat, bcat, ones, name):
    Bb, T, _ = h.shape
    nc = T // CHUNK
    fuse_in = inw is not None
    fuse_out = wcat is not None
    nout = wcat.shape[1] if fuse_out else BD
    full = lambda arr: pl.BlockSpec(arr.shape, lambda b, c: (0,) * arr.ndim)
    if not fuse_in:
        inw = jnp.zeros((1, 128), BF16)
        inb = jnp.zeros((1, 128), F32)
    if not fuse_out:
        wcat = jnp.zeros((1, 128), BF16)
        bcat = jnp.zeros((1, 128), F32)
    return pl.pallas_call(
        functools.partial(_layer_kernel, fuse_in, fuse_out),
        out_shape=jax.ShapeDtypeStruct((Bb, T, nout), F32),
        grid=(Bb, nc),
        in_specs=[
            pl.BlockSpec((1, CHUNK, h.shape[2]), lambda b, c: (b, c, 0)),
            full(inw), full(inb),
            full(wall), full(cw), full(cb),
            full(dtb), full(alog), full(dv), full(gnw), full(nw), full(outw),
            full(wcat), full(bcat),
            full(ones),
        ],
        out_specs=pl.BlockSpec((1, CHUNK, nout), lambda b, c: (b, c, 0)),
        scratch_shapes=[
            pltpu.VMEM((H_, NP, P_), F32),
            pltpu.VMEM((8, CONV), F32),
            pltpu.VMEM((CHUNK, I_), F32),
        ],
        compiler_params=pltpu.CompilerParams(
            dimension_semantics=("parallel", "arbitrary"),
            vmem_limit_bytes=50 * 1024 * 1024,
        ),
        name=name,
    )(h, inw, inb, wall, cw, cb, dtb, alog, dv, gnw, nw, outw,
      wcat, bcat, ones)


def kernel(x, in_w, in_b, norm_w, mix_in_w, conv_w, conv_b, dt_bias,
           A_log, D, gnorm_w, mix_out_w, out_w, out_b, code_w, code_b):
    Bb, T, IN = x.shape
    L = mix_in_w.shape[0]

    ones = jnp.ones((I_, 128), BF16)
    wmix = mix_in_w.astype(BF16)
    wout = mix_out_w.astype(BF16)
    wcat = jnp.concatenate([out_w, code_w], axis=1).astype(BF16)
    bcat = jnp.concatenate([out_b, code_b]).reshape(1, -1)

    h = x
    for l in range(L):
        h = _layer(
            h,
            in_w.astype(BF16) if l == 0 else None,
            in_b.reshape(1, BD) if l == 0 else None,
            wmix[l],
            conv_w[l][:, 0, :],
            conv_b[l].reshape(1, CONV),
            dt_bias[l].reshape(1, H_),
            A_log[l].reshape(1, H_),
            D[l].reshape(1, H_),
            gnorm_w[l].reshape(1, I_),
            norm_w[l].reshape(1, BD),
            wout[l],
            wcat if l == L - 1 else None,
            bcat if l == L - 1 else None,
            ones,
            f"mamba_layer_{l}",
        )
    no = out_w.shape[1]
    return (h[:, :, :no], h[:, :, no:])
